# Initial kernel scaffold; baseline (speedup 1.0000x reference)
import jax
import jax.numpy as jnp
from jax.experimental import pallas as pl


def kernel(features, edge_index, W1, b1, W2, b2, W3, b3):
    raise NotImplementedError("write your pallas kernel here")


# speedup vs baseline: 11.3968x; 11.3968x over previous
import functools
import jax
import jax.numpy as jnp
from jax import lax
from jax.experimental import pallas as pl
from jax.experimental.pallas import tpu as pltpu
from jax.experimental.pallas import tpu_sc as plsc

N = 10000
NP = 10112
NPD = 10240
E = 320000
D = 128
W3_PAD = 48

NC = 2
NS = 16
NW = NC * NS
CH = 128
NCHUNK = E // CH
CPT = NCHUNK // NW
XTRA = NCHUNK - CPT * NW
ROWS = NP // NS
ROWSD = NPD // NS
ZR = 128

_mesh = functools.partial(
    plsc.VectorSubcoreMesh, core_axis_name="c", subcore_axis_name="s",
    num_cores=NC, num_subcores=NS)


def _make_deg_kernel():
  @functools.partial(
      pl.kernel,
      out_type=jax.ShapeDtypeStruct((2 * 2 * NPD,), jnp.float32),
      mesh=_mesh(),
      scratch_types=[
          pltpu.VMEM((CH,), jnp.int32),
          pltpu.VMEM((CH,), jnp.int32),
          pltpu.VMEM((CH,), jnp.int32),
          pltpu.VMEM((CH,), jnp.int32),
          pltpu.VMEM((CH,), jnp.float32),
          pltpu.VMEM((ROWSD,), jnp.float32),
          pltpu.VMEM_SHARED((NPD,), jnp.float32),
          pltpu.VMEM_SHARED((NPD,), jnp.float32),
          pltpu.SemaphoreType.DMA,
          pltpu.SemaphoreType.DMA,
      ],
  )
  def deg_kernel(src_hbm, dst_hbm, ones_hbm, zeros_hbm, out_hbm,
                 sa, da, sb, db, ones_v, zbuf, hist_s, hist_d, semA, semB):
    c = lax.axis_index("c")
    s = lax.axis_index("s")
    t = c * NS + s
    r0 = t * CPT + jnp.minimum(t, XTRA)
    extra = t < XTRA
    pltpu.sync_copy(ones_hbm, ones_v)
    pltpu.sync_copy(zeros_hbm, zbuf)
    row0 = s * ROWSD
    pltpu.sync_copy(zbuf, hist_s.at[pl.ds(row0, ROWSD)])
    pltpu.sync_copy(zbuf, hist_d.at[pl.ds(row0, ROWSD)])
    plsc.subcore_barrier()

    def eb(ch):
      return (r0 + ch) * CH

    def load(ch, sref, dref, sem):
      pltpu.async_copy(src_hbm.at[pl.ds(eb(ch), CH)], sref, sem)
      pltpu.async_copy(dst_hbm.at[pl.ds(eb(ch), CH)], dref, sem)

    def wait(ch, sref, dref, sem):
      pltpu.make_async_copy(src_hbm.at[pl.ds(eb(ch), CH)], sref, sem).wait()
      pltpu.make_async_copy(dst_hbm.at[pl.ds(eb(ch), CH)], dref, sem).wait()

    def scat(sref, dref):
      pltpu.sync_copy(ones_v, hist_s.at[sref], add=True)
      pltpu.sync_copy(ones_v, hist_d.at[dref], add=True)

    load(0, sa, da, semA)
    load(1, sb, db, semB)

    def body(j, carry):
      a = 2 * j
      wait(a, sa, da, semA)
      scat(sa, da)

      @pl.when(a + 2 < CPT)
      def _():
        load(a + 2, sa, da, semA)

      wait(a + 1, sb, db, semB)
      scat(sb, db)

      @pl.when(a + 3 < CPT)
      def _():
        load(a + 3, sb, db, semB)

      return carry

    lax.fori_loop(0, CPT // 2, body, 0)

    @pl.when(extra)
    def _():
      load(CPT, sa, da, semA)
      wait(CPT, sa, da, semA)
      scat(sa, da)

    plsc.subcore_barrier()
    pltpu.sync_copy(hist_s.at[pl.ds(row0, ROWSD)],
                    out_hbm.at[pl.ds((c * 2 + 0) * NPD + row0, ROWSD)])
    pltpu.sync_copy(hist_d.at[pl.ds(row0, ROWSD)],
                    out_hbm.at[pl.ds((c * 2 + 1) * NPD + row0, ROWSD)])

  return deg_kernel


def _make_agg_kernel(width):
  @functools.partial(
      pl.kernel,
      out_type=jax.ShapeDtypeStruct((2 * NP, width), jnp.float32),
      mesh=_mesh(),
      scratch_types=[
          pltpu.VMEM((CH,), jnp.int32),
          pltpu.VMEM((CH,), jnp.int32),
          pltpu.VMEM((CH,), jnp.int32),
          pltpu.VMEM((CH,), jnp.int32),
          pltpu.VMEM((CH, width), jnp.float32),
          pltpu.VMEM((CH, width), jnp.float32),
          pltpu.VMEM((ZR, width), jnp.float32),
          pltpu.VMEM_SHARED((NP, width), jnp.float32),
          pltpu.SemaphoreType.DMA,
          pltpu.SemaphoreType.DMA,
          pltpu.SemaphoreType.DMA,
          pltpu.SemaphoreType.DMA,
      ],
  )
  def agg_kernel(h_hbm, src_hbm, dst_hbm, zeros_hbm, out_hbm,
                 sa, da, sb, db, rows0, rows1, zbuf, agg,
                 semIA, semIB, semG0, semG1):
    c = lax.axis_index("c")
    s = lax.axis_index("s")
    t = c * NS + s
    r0 = t * CPT + jnp.minimum(t, XTRA)
    extra = t < XTRA
    row0 = s * ROWS
    pltpu.sync_copy(zeros_hbm, zbuf)
    for z in range(ROWS // ZR):
      pltpu.sync_copy(zbuf, agg.at[pl.ds(row0 + z * ZR, ZR)])
    rem = ROWS % ZR
    if rem:
      pltpu.sync_copy(zbuf.at[pl.ds(0, rem)],
                      agg.at[pl.ds(row0 + (ROWS // ZR) * ZR, rem)])
    plsc.subcore_barrier()

    def eb(ch):
      return (r0 + ch) * CH

    def load(ch, sref, dref, sem):
      pltpu.async_copy(src_hbm.at[pl.ds(eb(ch), CH)], sref, sem)
      pltpu.async_copy(dst_hbm.at[pl.ds(eb(ch), CH)], dref, sem)

    def wait_idx(ch, sref, dref, sem):
      pltpu.make_async_copy(src_hbm.at[pl.ds(eb(ch), CH)], sref, sem).wait()
      pltpu.make_async_copy(dst_hbm.at[pl.ds(eb(ch), CH)], dref, sem).wait()

    load(0, sa, da, semIA)
    wait_idx(0, sa, da, semIA)
    pltpu.async_copy(h_hbm.at[sa], rows0, semG0)
    load(1, sb, db, semIB)

    def body(j, carry):
      a = 2 * j
      wait_idx(a + 1, sb, db, semIB)
      pltpu.async_copy(h_hbm.at[sb], rows1, semG1)
      pltpu.make_async_copy(h_hbm.at[sa], rows0, semG0).wait()
      pltpu.sync_copy(rows0, agg.at[da], add=True)

      @pl.when(a + 2 < CPT)
      def _():
        load(a + 2, sa, da, semIA)
        wait_idx(a + 2, sa, da, semIA)
        pltpu.async_copy(h_hbm.at[sa], rows0, semG0)

      pltpu.make_async_copy(h_hbm.at[sb], rows1, semG1).wait()
      pltpu.sync_copy(rows1, agg.at[db], add=True)

      @pl.when(a + 3 < CPT)
      def _():
        load(a + 3, sb, db, semIB)

      return carry

    lax.fori_loop(0, CPT // 2, body, 0)

    @pl.when(extra)
    def _():
      load(CPT, sa, da, semIA)
      wait_idx(CPT, sa, da, semIA)
      pltpu.async_copy(h_hbm.at[sa], rows0, semG0)
      pltpu.make_async_copy(h_hbm.at[sa], rows0, semG0).wait()
      pltpu.sync_copy(rows0, agg.at[da], add=True)

    plsc.subcore_barrier()
    pltpu.sync_copy(agg.at[pl.ds(row0, ROWS)],
                    out_hbm.at[pl.ds(c * NP + row0, ROWS)])

  return agg_kernel


BN = NP


def _prep_body(degp_ref, feat_ref, ns_ref, nd_ref, h0_ref):
  p = degp_ref[...]
  deg_out = p[0:1] + p[2:3]
  deg_in = p[1:2] + p[3:4]
  ns = jnp.transpose(lax.rsqrt(jnp.where(deg_out > 0, deg_out, 1.0)))
  nd = jnp.transpose(lax.rsqrt(jnp.where(deg_in > 0, deg_in, 1.0)))
  ns_ref[...] = ns
  nd_ref[...] = nd
  h0_ref[...] = feat_ref[...] * ns


def _prep(degp, features):
  return pl.pallas_call(
      _prep_body,
      grid=(NP // BN,),
      in_specs=[
          pl.BlockSpec((4, BN), lambda i: (0, i)),
          pl.BlockSpec((BN, D), lambda i: (i, 0)),
      ],
      out_specs=[
          pl.BlockSpec((BN, 1), lambda i: (i, 0)),
          pl.BlockSpec((BN, 1), lambda i: (i, 0)),
          pl.BlockSpec((BN, D), lambda i: (i, 0)),
      ],
      out_shape=[
          jax.ShapeDtypeStruct((NP, 1), jnp.float32),
          jax.ShapeDtypeStruct((NP, 1), jnp.float32),
          jax.ShapeDtypeStruct((NP, D), jnp.float32),
      ],
  )(degp, features)


def _mid_body(a_ref, nd_ref, ns_ref, w_ref, b_ref, o_ref):
  x = (a_ref[0] + a_ref[1]) * nd_ref[...]
  y = jnp.dot(x, w_ref[...], preferred_element_type=jnp.float32) + b_ref[...]
  o_ref[...] = jnp.maximum(y, 0.0) * ns_ref[...]


def _mid_layer(a, nd, ns, w, b):
  return pl.pallas_call(
      _mid_body,
      grid=(NP // BN,),
      in_specs=[
          pl.BlockSpec((2, BN, D), lambda i: (0, i, 0)),
          pl.BlockSpec((BN, 1), lambda i: (i, 0)),
          pl.BlockSpec((BN, 1), lambda i: (i, 0)),
          pl.BlockSpec((D, D), lambda i: (0, 0)),
          pl.BlockSpec((1, D), lambda i: (0, 0)),
      ],
      out_specs=pl.BlockSpec((BN, D), lambda i: (i, 0)),
      out_shape=jax.ShapeDtypeStruct((NP, D), jnp.float32),
  )(a, nd, ns, w, b)


def _final_body(a_ref, nd_ref, w3_ref, b_ref, o_ref):
  x = (a_ref[0] + a_ref[1]) * nd_ref[...]
  o_ref[...] = jnp.dot(
      x, w3_ref[...], preferred_element_type=jnp.float32) + b_ref[...]


def _final_layer(a, nd, w3p, b3p):
  return pl.pallas_call(
      _final_body,
      grid=(NP // BN,),
      in_specs=[
          pl.BlockSpec((2, BN, D), lambda i: (0, i, 0)),
          pl.BlockSpec((BN, 1), lambda i: (i, 0)),
          pl.BlockSpec((D, W3_PAD), lambda i: (0, 0)),
          pl.BlockSpec((1, W3_PAD), lambda i: (0, 0)),
      ],
      out_specs=pl.BlockSpec((BN, W3_PAD), lambda i: (i, 0)),
      out_shape=jax.ShapeDtypeStruct((NP, W3_PAD), jnp.float32),
  )(a, nd, w3p, b3p)


_sc_cache = {}


def _sc(name):
  if not _sc_cache:
    _sc_cache["deg"] = _make_deg_kernel()
    _sc_cache["agg128"] = _make_agg_kernel(D)
  return _sc_cache[name]


@jax.jit
def kernel(features, edge_index, W1, b1, W2, b2, W3, b3):
  src = edge_index[0]
  dst = edge_index[1]
  nc = W3.shape[1]
  w3p = jnp.pad(W3, ((0, 0), (0, W3_PAD - nc)))
  b3p = jnp.pad(b3, (0, W3_PAD - nc)).reshape(1, W3_PAD)
  b1r = b1.reshape(1, D)
  b2r = b2.reshape(1, D)
  ones1 = jnp.ones((CH,), jnp.float32)
  zeros1 = jnp.zeros((ROWSD,), jnp.float32)
  zeros128 = jnp.zeros((ZR, D), jnp.float32)
  feat_p = jnp.pad(features, ((0, NP - N), (0, 0)))

  degp = _sc("deg")(src, dst, ones1, zeros1).reshape(4, NPD)[:, :NP]
  ns, nd, h0 = _prep(degp, feat_p)

  a1 = _sc("agg128")(h0, src, dst, zeros128).reshape(2, NP, D)
  h1 = _mid_layer(a1, nd, ns, W1, b1r)

  a2 = _sc("agg128")(h1, src, dst, zeros128).reshape(2, NP, D)
  h2 = _mid_layer(a2, nd, ns, W2, b2r)

  a3 = _sc("agg128")(h2, src, dst, zeros128).reshape(2, NP, D)
  out = _final_layer(a3, nd, w3p, b3p)
  return out[:N, :nc]
